# initial kernel scaffold (unmeasured)
import jax
import jax.numpy as jnp
from jax import lax
from jax.experimental import pallas as pl
from jax.experimental.pallas import tpu as pltpu


def kernel(x, dest):
    m, n = x.shape
    me = lax.axis_index("y")
    is_send = (dest != me).astype(jnp.int32)
    ns = jnp.sum(is_send).astype(jnp.int32)
    nk = jnp.int32(m) - ns

    x_ks = x[jnp.argsort(is_send, stable=True)]
    x_sk = x[jnp.argsort(1 - is_send, stable=True)]

    is0 = me == 0
    local_buf = jnp.where(is0, x_sk, x_ks)
    send_buf = jnp.where(is0, x_ks, x_sk)
    l_off = jnp.where(is0, m - ns, m + ns).astype(jnp.int32)
    r_off = jnp.where(is0, m - nk, m + nk).astype(jnp.int32)
    offs = jnp.stack([l_off, r_off])

    def body(off_ref, local_ref, send_ref, out_ref, staging, send_sem, recv_sem):
        my_x = lax.axis_index("x")
        my_y = lax.axis_index("y")
        my_z = lax.axis_index("z")
        peer = (my_x, 1 - my_y, my_z)

        barrier_sem = pltpu.get_barrier_semaphore()
        pl.semaphore_signal(
            barrier_sem, inc=1, device_id=peer,
            device_id_type=pl.DeviceIdType.MESH,
        )
        pl.semaphore_wait(barrier_sem, 1)

        rdma = pltpu.make_async_remote_copy(
            src_ref=send_ref,
            dst_ref=staging.at[pl.ds(off_ref[1], m)],
            send_sem=send_sem,
            recv_sem=recv_sem,
            device_id=peer,
            device_id_type=pl.DeviceIdType.MESH,
        )
        rdma.start()
        staging[pl.ds(off_ref[0], m), :] = local_ref[:, :]
        rdma.wait()
        out_ref[:, :] = staging[pl.ds(m, m), :]

    return pl.pallas_call(
        body,
        out_shape=jax.ShapeDtypeStruct((m, n), x.dtype),
        in_specs=[
            pl.BlockSpec(memory_space=pltpu.SMEM),
            pl.BlockSpec(memory_space=pltpu.VMEM),
            pl.BlockSpec(memory_space=pltpu.VMEM),
        ],
        out_specs=pl.BlockSpec(memory_space=pltpu.VMEM),
        scratch_shapes=[
            pltpu.VMEM((3 * m, n), x.dtype),
            pltpu.SemaphoreType.DMA,
            pltpu.SemaphoreType.DMA,
        ],
        compiler_params=pltpu.CompilerParams(collective_id=0),
    )(offs, local_buf, send_buf)


# baseline (device time: 23201 ns/iter reference)
import jax
import jax.numpy as jnp
from jax import lax
from jax.experimental import pallas as pl
from jax.experimental.pallas import tpu as pltpu


def kernel(x, dest):
    m, n = x.shape
    me = lax.axis_index("y")
    is_send = (dest != me).astype(jnp.int32)
    ns = jnp.sum(is_send).astype(jnp.int32)
    nk = jnp.int32(m) - ns

    x_ks = x[jnp.argsort(is_send, stable=True)]
    x_sk = x[jnp.argsort(1 - is_send, stable=True)]
    cnt = jnp.stack([nk, ns])

    def body(cnt_ref, local_ref, send_ref, out_ref, recv_buf, send_sem, recv_sem):
        my_x = lax.axis_index("x")
        my_y = lax.axis_index("y")
        my_z = lax.axis_index("z")
        peer = (my_x, 1 - my_y, my_z)

        barrier_sem = pltpu.get_barrier_semaphore()
        pl.semaphore_signal(
            barrier_sem, inc=1, device_id=peer,
            device_id_type=pl.DeviceIdType.MESH,
        )
        pl.semaphore_wait(barrier_sem, 1)

        rdma = pltpu.make_async_remote_copy(
            src_ref=send_ref,
            dst_ref=recv_buf,
            send_sem=send_sem,
            recv_sem=recv_sem,
            device_id=peer,
            device_id_type=pl.DeviceIdType.MESH,
        )
        rdma.start()
        rdma.wait()

        is0 = my_y == 0
        cut = jnp.where(is0, cnt_ref[0], cnt_ref[1])
        local = local_ref[:, :]
        recv = recv_buf[:, :]
        low = jnp.where(is0, local, recv)
        high = jnp.where(is0, recv, local)
        rows = lax.broadcasted_iota(jnp.int32, (m, n), 0)
        out_ref[:, :] = jnp.where(
            rows < cut, low, pltpu.roll(high, cut, axis=0)
        )

    return pl.pallas_call(
        body,
        out_shape=jax.ShapeDtypeStruct((m, n), x.dtype),
        in_specs=[
            pl.BlockSpec(memory_space=pltpu.SMEM),
            pl.BlockSpec(memory_space=pltpu.VMEM),
            pl.BlockSpec(memory_space=pltpu.VMEM),
        ],
        out_specs=pl.BlockSpec(memory_space=pltpu.VMEM),
        scratch_shapes=[
            pltpu.VMEM((m, n), x.dtype),
            pltpu.SemaphoreType.DMA,
            pltpu.SemaphoreType.DMA,
        ],
        compiler_params=pltpu.CompilerParams(collective_id=0),
    )(cnt, x_ks, x_sk)


# device time: 18033 ns/iter; 1.2866x vs baseline; 1.2866x over previous
import jax
import jax.numpy as jnp
from jax import lax
from jax.experimental import pallas as pl
from jax.experimental.pallas import tpu as pltpu


def kernel(x, dest):
    m, n = x.shape
    me = lax.axis_index("y")
    is_send = (dest != me).astype(jnp.int32)
    ns = jnp.sum(is_send).astype(jnp.int32)
    nk = jnp.int32(m) - ns

    x_ks = x[jnp.argsort(is_send, stable=True)]
    cnt = jnp.stack([nk, ns])

    def body(cnt_ref, x_ref, out_ref, recv_buf, send_sem, recv_sem):
        my_x = lax.axis_index("x")
        my_y = lax.axis_index("y")
        my_z = lax.axis_index("z")
        peer = (my_x, 1 - my_y, my_z)

        barrier_sem = pltpu.get_barrier_semaphore()
        pl.semaphore_signal(
            barrier_sem, inc=1, device_id=peer,
            device_id_type=pl.DeviceIdType.MESH,
        )
        pl.semaphore_wait(barrier_sem, 1)

        rdma = pltpu.make_async_remote_copy(
            src_ref=x_ref,
            dst_ref=recv_buf,
            send_sem=send_sem,
            recv_sem=recv_sem,
            device_id=peer,
            device_id_type=pl.DeviceIdType.MESH,
        )
        rdma.start()
        rdma.wait()

        nk_ = cnt_ref[0]
        ns_ = cnt_ref[1]
        rows = lax.broadcasted_iota(jnp.int32, (m, n), 0)
        sel = jnp.where(rows < nk_, x_ref[:, :], recv_buf[:, :])
        shift = jnp.where(my_y == 0, 0, ns_)
        out_ref[:, :] = pltpu.roll(sel, shift, axis=0)

    return pl.pallas_call(
        body,
        out_shape=jax.ShapeDtypeStruct((m, n), x.dtype),
        in_specs=[
            pl.BlockSpec(memory_space=pltpu.SMEM),
            pl.BlockSpec(memory_space=pltpu.VMEM),
        ],
        out_specs=pl.BlockSpec(memory_space=pltpu.VMEM),
        scratch_shapes=[
            pltpu.VMEM((m, n), x.dtype),
            pltpu.SemaphoreType.DMA,
            pltpu.SemaphoreType.DMA,
        ],
        compiler_params=pltpu.CompilerParams(collective_id=0),
    )(cnt, x_ks)


# device time: 11871 ns/iter; 1.9544x vs baseline; 1.5191x over previous
import jax
import jax.numpy as jnp
from jax import lax
from jax.experimental import pallas as pl
from jax.experimental.pallas import tpu as pltpu

_CHUNK = 64


def kernel(x, dest):
    m, n = x.shape
    me = lax.axis_index("y")
    is_send = (dest != me).astype(jnp.int32)
    ns = jnp.sum(is_send)
    nk = jnp.int32(m) - ns

    cs = jnp.cumsum(is_send)
    idx = jnp.arange(m, dtype=jnp.int32)
    pos = jnp.where(is_send == 1, nk + cs - 1, idx - cs)
    cnt = jnp.stack([nk, ns])

    n_chunks = m // _CHUNK

    def body(cnt_ref, pos_ref, x_ref, out_ref, xks_buf, recv_buf,
             send_sems, recv_sems):
        my_x = lax.axis_index("x")
        my_y = lax.axis_index("y")
        my_z = lax.axis_index("z")
        peer = (my_x, 1 - my_y, my_z)
        nk_ = cnt_ref[0]
        ns_ = cnt_ref[1]

        barrier_sem = pltpu.get_barrier_semaphore()
        pl.semaphore_signal(
            barrier_sem, inc=1, device_id=peer,
            device_id_type=pl.DeviceIdType.MESH,
        )
        pl.semaphore_wait(barrier_sem, 1)

        j_iota = lax.broadcasted_iota(jnp.int32, (m, m), 0)
        p_mat = (j_iota == pos_ref[:, :]).astype(jnp.float32)
        xks_buf[:, :] = jax.lax.dot(
            p_mat, x_ref[:, :], preferred_element_type=jnp.float32
        )

        def chunk_rdma(c):
            return pltpu.make_async_remote_copy(
                src_ref=xks_buf.at[pl.ds(c * _CHUNK, _CHUNK)],
                dst_ref=recv_buf.at[pl.ds(c * _CHUNK, _CHUNK)],
                send_sem=send_sems.at[c],
                recv_sem=recv_sems.at[c],
                device_id=peer,
                device_id_type=pl.DeviceIdType.MESH,
            )

        def pred(c):
            if c == n_chunks - 1:
                return nk_ >= 0
            return (c + 1) * _CHUNK > nk_

        for c in range(n_chunks):
            @pl.when(pred(c))
            def _(c=c):
                chunk_rdma(c).start()

        for c in range(n_chunks):
            @pl.when(pred(c))
            def _(c=c):
                chunk_rdma(c).wait()

        rows = lax.broadcasted_iota(jnp.int32, (m, n), 0)
        sel = jnp.where(rows < nk_, xks_buf[:, :], recv_buf[:, :])
        shift = jnp.where(my_y == 0, 0, ns_)
        out_ref[:, :] = pltpu.roll(sel, shift, axis=0)

    return pl.pallas_call(
        body,
        out_shape=jax.ShapeDtypeStruct((m, n), x.dtype),
        in_specs=[
            pl.BlockSpec(memory_space=pltpu.SMEM),
            pl.BlockSpec(memory_space=pltpu.VMEM),
            pl.BlockSpec(memory_space=pltpu.VMEM),
        ],
        out_specs=pl.BlockSpec(memory_space=pltpu.VMEM),
        scratch_shapes=[
            pltpu.VMEM((m, n), x.dtype),
            pltpu.VMEM((m, n), x.dtype),
            pltpu.SemaphoreType.DMA((n_chunks,)),
            pltpu.SemaphoreType.DMA((n_chunks,)),
        ],
        compiler_params=pltpu.CompilerParams(collective_id=0),
    )(cnt, pos.reshape(1, m), x)


# device time: 11675 ns/iter; 1.9872x vs baseline; 1.0168x over previous
import jax
import jax.numpy as jnp
from jax import lax
from jax.experimental import pallas as pl
from jax.experimental.pallas import tpu as pltpu

_CHUNK = 32


def kernel(x, dest):
    m, n = x.shape
    me = lax.axis_index("y")
    is_send = (dest != me).astype(jnp.int32)
    ns = jnp.sum(is_send)
    nk = jnp.int32(m) - ns
    cnt = jnp.stack([nk, ns])

    n_chunks = m // _CHUNK

    def body(cnt_ref, snd_ref, x_ref, out_ref, xks_buf, recv_buf,
             send_sems, recv_sems):
        my_x = lax.axis_index("x")
        my_y = lax.axis_index("y")
        my_z = lax.axis_index("z")
        peer = (my_x, 1 - my_y, my_z)
        nk_ = cnt_ref[0]
        ns_ = cnt_ref[1]

        barrier_sem = pltpu.get_barrier_semaphore()
        pl.semaphore_signal(
            barrier_sem, inc=1, device_id=peer,
            device_id_type=pl.DeviceIdType.MESH,
        )
        pl.semaphore_wait(barrier_sem, 1)

        vf = snd_ref[:, :].astype(jnp.float32)
        k_i = lax.broadcasted_iota(jnp.int32, (m, m), 0)
        i_i = lax.broadcasted_iota(jnp.int32, (m, m), 1)
        tri = (k_i <= i_i).astype(jnp.float32)
        cs = jax.lax.dot(vf, tri, preferred_element_type=jnp.float32)
        nsv = cs[:, m - 1 : m]
        idxr = lax.broadcasted_iota(jnp.int32, (1, m), 1).astype(jnp.float32)
        pos = jnp.where(vf == 1.0, (m - nsv) + cs - 1.0, idxr - cs)

        j_f = lax.broadcasted_iota(jnp.int32, (m, m), 0).astype(jnp.float32)
        p_mat = (j_f == pos).astype(jnp.float32)
        xks_buf[:, :] = jax.lax.dot(
            p_mat, x_ref[:, :], preferred_element_type=jnp.float32
        )

        def chunk_rdma(c):
            return pltpu.make_async_remote_copy(
                src_ref=xks_buf.at[pl.ds(c * _CHUNK, _CHUNK)],
                dst_ref=recv_buf.at[pl.ds(c * _CHUNK, _CHUNK)],
                send_sem=send_sems.at[c],
                recv_sem=recv_sems.at[c],
                device_id=peer,
                device_id_type=pl.DeviceIdType.MESH,
            )

        def pred(c):
            if c == n_chunks - 1:
                return nk_ >= 0
            return (c + 1) * _CHUNK > nk_

        for c in range(n_chunks):
            @pl.when(pred(c))
            def _(c=c):
                chunk_rdma(c).start()

        for c in range(n_chunks):
            @pl.when(pred(c))
            def _(c=c):
                chunk_rdma(c).wait()

        rows = lax.broadcasted_iota(jnp.int32, (m, n), 0)
        sel = jnp.where(rows < nk_, xks_buf[:, :], recv_buf[:, :])
        shift = jnp.where(my_y == 0, 0, ns_)
        out_ref[:, :] = pltpu.roll(sel, shift, axis=0)

    return pl.pallas_call(
        body,
        out_shape=jax.ShapeDtypeStruct((m, n), x.dtype),
        in_specs=[
            pl.BlockSpec(memory_space=pltpu.SMEM),
            pl.BlockSpec(memory_space=pltpu.VMEM),
            pl.BlockSpec(memory_space=pltpu.VMEM),
        ],
        out_specs=pl.BlockSpec(memory_space=pltpu.VMEM),
        scratch_shapes=[
            pltpu.VMEM((m, n), x.dtype),
            pltpu.VMEM((m, n), x.dtype),
            pltpu.SemaphoreType.DMA((n_chunks,)),
            pltpu.SemaphoreType.DMA((n_chunks,)),
        ],
        compiler_params=pltpu.CompilerParams(collective_id=0),
    )(cnt, is_send.reshape(1, m), x)
